# baseline (speedup 1.0000x reference)
import functools

import jax
import jax.numpy as jnp
from jax import lax
from jax.experimental import pallas as pl
from jax.experimental.pallas import tpu as pltpu
from jax.experimental.pallas import tpu_sc as plsc

B = 1024
L = 200
DIM = 32
N = B * L

NC = 2
NS = 16
NW = NC * NS
B_PER_W = N // NW
CHUNK = 3200
N_CHUNKS = B_PER_W // CHUNK


@functools.partial(
    pl.kernel,
    out_type=jax.ShapeDtypeStruct((N, DIM), jnp.float32),
    mesh=plsc.VectorSubcoreMesh(core_axis_name="c", subcore_axis_name="s"),
    scratch_types=[
        pltpu.VMEM((CHUNK,), jnp.int32),
        pltpu.VMEM((CHUNK, DIM), jnp.float32),
        pltpu.SemaphoreType.DMA,
    ],
    compiler_params=pltpu.CompilerParams(use_tc_tiling_on_sc=False),
)
def _gather_kernel(ids_hbm, table_hbm, out_hbm, idx_v, rows_v, sem):
    wid = lax.axis_index("s") * NC + lax.axis_index("c")
    base = wid * B_PER_W
    for i in range(N_CHUNKS):
        off = base + i * CHUNK
        pltpu.sync_copy(ids_hbm.at[pl.ds(off, CHUNK)], idx_v)
        pltpu.async_copy(table_hbm.at[idx_v], rows_v, sem).wait()
        pltpu.sync_copy(rows_v, out_hbm.at[pl.ds(off, CHUNK)])


def kernel(input_ids, attention_mask, table):
    ids_flat = input_ids.reshape(N)
    rows = _gather_kernel(ids_flat, table)
    return rows.reshape(B, L, DIM), attention_mask


# speedup vs baseline: 1.0479x; 1.0479x over previous
import functools

import jax
import jax.numpy as jnp
from jax import lax
from jax.experimental import pallas as pl
from jax.experimental.pallas import tpu as pltpu
from jax.experimental.pallas import tpu_sc as plsc

B = 1024
L = 200
DIM = 32
V = 1000000
N = B * L

NC = 2
NS = 16
NW = NC * NS

BLK = 2048
G1 = (V + BLK - 1) // BLK
V4 = G1 * (BLK // 4)
PIECE = 256
K2_STEPS = (L + NW - 1) // NW


@functools.partial(
    pl.pallas_call,
    grid=(G1,),
    in_specs=[pl.BlockSpec((DIM, BLK), lambda i: (0, i))],
    out_specs=pl.BlockSpec((BLK // 4, 128), lambda i: (i, 0)),
    out_shape=jax.ShapeDtypeStruct((V4, 128), jnp.float32),
)
def _repack(x_ref, o_ref):
    x = x_ref[...]
    o_ref[...] = jnp.concatenate(
        [x[:, 512 * a:512 * a + 512].T for a in range(4)], axis=1
    )


@functools.partial(
    pl.kernel,
    out_type=jax.ShapeDtypeStruct((L, PIECE, 4, DIM), jnp.float32),
    mesh=plsc.VectorSubcoreMesh(core_axis_name="c", subcore_axis_name="s"),
    scratch_types=[
        pltpu.VMEM((B,), jnp.int32),
        pltpu.VMEM((B,), jnp.int32),
        pltpu.VMEM((PIECE, DIM), jnp.float32),
        pltpu.SemaphoreType.DMA,
    ],
    compiler_params=pltpu.CompilerParams(use_tc_tiling_on_sc=False),
)
def _gather(idsT_hbm, rm_hbm, out_hbm, ids_v, qm_v, buf, sem):
    wid = lax.axis_index("s") * NC + lax.axis_index("c")

    def do_l(l):
        pltpu.sync_copy(idsT_hbm.at[l], ids_v)

        def qbody(g, _):
            v = ids_v[pl.ds(g * 16, 16)]
            hi = lax.bitwise_and(v, jnp.full((16,), -2048, jnp.int32))
            mid = lax.shift_left(
                lax.bitwise_and(v, jnp.full((16,), 511, jnp.int32)),
                jnp.full((16,), 2, jnp.int32),
            )
            lo = lax.bitwise_and(
                lax.shift_right_logical(v, jnp.full((16,), 9, jnp.int32)),
                jnp.full((16,), 3, jnp.int32),
            )
            qm_v[pl.ds(g * 16, 16)] = lax.bitwise_or(hi, lax.bitwise_or(mid, lo))
            return 0

        lax.fori_loop(0, B // 16, qbody, 0)

        for a in range(4):
            pltpu.async_copy(
                rm_hbm.at[qm_v.at[pl.ds(a * PIECE, PIECE)]], buf, sem
            ).wait()
            pltpu.sync_copy(buf, out_hbm.at[l, :, a, :])

    for k in range(K2_STEPS):
        l = wid + NW * k
        if k < K2_STEPS - 1:
            do_l(l)
        else:
            @pl.when(l < L)
            def _():
                do_l(l)


@functools.partial(
    pl.pallas_call,
    grid=(L,),
    in_specs=[pl.BlockSpec((PIECE, 128), lambda i: (i, 0))],
    out_specs=pl.BlockSpec((1, DIM, B), lambda i: (i, 0, 0)),
    out_shape=jax.ShapeDtypeStruct((L, DIM, B), jnp.float32),
)
def _slabify(x_ref, o_ref):
    x = x_ref[...]
    s = jnp.concatenate([x[:, 32 * a:32 * a + 32].T for a in range(4)], axis=1)
    o_ref[...] = s[None]


def kernel(input_ids, attention_mask, table):
    rm = _repack(table.T)
    x2 = _gather(input_ids.T, rm.reshape(V4 * 4, DIM))
    outT = _slabify(x2.reshape(L * PIECE, 128))
    return jnp.transpose(outT, (2, 0, 1)), attention_mask
